# baseline (speedup 1.0000x reference)
import functools

import jax
import jax.numpy as jnp
from jax import lax
from jax.experimental import pallas as pl
from jax.experimental.pallas import tpu as pltpu
from jax.experimental.pallas import tpu_sc as plsc

NC = 2
NS = 16
CHUNK = 128


def _round_up(a, b):
    return (a + b - 1) // b * b


KB = 8


def _sc_aggregate(data, colh, rowh, zeros2d, zeros1d, ones1, n_pad, nchunk):
    D = data.shape[1]
    rpt = n_pad // NS
    nw = nchunk // KB
    mesh = plsc.VectorSubcoreMesh(core_axis_name="c", subcore_axis_name="s")

    @functools.partial(
        pl.kernel,
        out_type=(
            jax.ShapeDtypeStruct((NC, n_pad, D), jnp.float32),
            jax.ShapeDtypeStruct((NC, n_pad), jnp.float32),
        ),
        mesh=mesh,
        scratch_types=[
            pltpu.VMEM((2, KB, CHUNK), jnp.int32),
            pltpu.VMEM((2, KB, CHUNK), jnp.int32),
            pltpu.VMEM((2, CHUNK, D), jnp.float32),
            pltpu.VMEM((CHUNK,), jnp.float32),
            pltpu.VMEM_SHARED((n_pad, D), jnp.float32),
            pltpu.VMEM_SHARED((n_pad,), jnp.float32),
            pltpu.SemaphoreType.DMA,
            pltpu.SemaphoreType.DMA,
        ],
    )
    def k(data_hbm, col_hbm, row_hbm, z2_hbm, z1_hbm, ones_hbm,
          agg_out, deg_out, col_w, row_w, gbuf, ones_v, sh_agg, sh_deg,
          gsem, isem):
        c = lax.axis_index("c")
        s = lax.axis_index("s")
        rbase = s * rpt
        pltpu.sync_copy(z2_hbm, sh_agg.at[pl.ds(rbase, rpt)])
        pltpu.sync_copy(z1_hbm, sh_deg.at[pl.ds(rbase, rpt)])
        pltpu.sync_copy(ones_hbm, ones_v)
        pltpu.async_copy(col_hbm.at[c, s, pl.ds(0, KB)], col_w.at[0], isem)
        pltpu.async_copy(row_hbm.at[c, s, pl.ds(0, KB)], row_w.at[0], isem)
        plsc.subcore_barrier()

        @pl.loop(0, nw, step=2)
        def _(ww):
            for slot in range(2):
                w = ww + slot
                pltpu.make_async_copy(
                    col_hbm.at[c, s, pl.ds(w * KB, KB)], col_w.at[slot], isem
                ).wait()
                pltpu.make_async_copy(
                    row_hbm.at[c, s, pl.ds(w * KB, KB)], row_w.at[slot], isem
                ).wait()

                @pl.when(w + 1 < nw)
                def _():
                    nb = (w + 1) * KB
                    pltpu.async_copy(col_hbm.at[c, s, pl.ds(nb, KB)],
                                     col_w.at[1 - slot], isem)
                    pltpu.async_copy(row_hbm.at[c, s, pl.ds(nb, KB)],
                                     row_w.at[1 - slot], isem)

                @pl.loop(0, KB, step=2)
                def _(jj):
                    da = pltpu.async_copy(
                        data_hbm.at[col_w.at[slot, jj]], gbuf.at[0], gsem)
                    db = pltpu.async_copy(
                        data_hbm.at[col_w.at[slot, jj + 1]], gbuf.at[1], gsem)
                    da.wait()
                    pltpu.sync_copy(gbuf.at[0],
                                    sh_agg.at[row_w.at[slot, jj]], add=True)
                    pltpu.sync_copy(ones_v,
                                    sh_deg.at[row_w.at[slot, jj]], add=True)
                    db.wait()
                    pltpu.sync_copy(gbuf.at[1],
                                    sh_agg.at[row_w.at[slot, jj + 1]], add=True)
                    pltpu.sync_copy(ones_v,
                                    sh_deg.at[row_w.at[slot, jj + 1]], add=True)

        plsc.subcore_barrier()
        pltpu.sync_copy(sh_agg.at[pl.ds(rbase, rpt)], agg_out.at[c, pl.ds(rbase, rpt)])
        pltpu.sync_copy(sh_deg.at[pl.ds(rbase, rpt)], deg_out.at[c, pl.ds(rbase, rpt)])

    return k(data, colh, rowh, zeros2d, zeros1d, ones1)


def _tc_linear(parts, degs3, W, b, relu):
    n_pad, D = parts.shape[1], parts.shape[2]
    H = W.shape[1]
    BLK = 512

    def body(p_ref, d_ref, w_ref, b_ref, o_ref):
        agg = p_ref[0] + p_ref[1]
        deg = d_ref[0] + d_ref[1]
        dinv = jnp.where(deg > 0, lax.rsqrt(jnp.maximum(deg, 1.0)), 0.0)
        out = jnp.dot(agg * dinv, w_ref[...],
                      preferred_element_type=jnp.float32,
                      precision=lax.Precision.HIGHEST)
        out = out + b_ref[...]
        if relu:
            out = jnp.maximum(out, 0.0)
        o_ref[...] = out

    return pl.pallas_call(
        body,
        grid=(n_pad // BLK,),
        in_specs=[
            pl.BlockSpec((NC, BLK, D), lambda i: (0, i, 0)),
            pl.BlockSpec((NC, BLK, 1), lambda i: (0, i, 0)),
            pl.BlockSpec((D, H), lambda i: (0, 0)),
            pl.BlockSpec((1, H), lambda i: (0, 0)),
        ],
        out_specs=pl.BlockSpec((BLK, H), lambda i: (i, 0)),
        out_shape=jax.ShapeDtypeStruct((n_pad, H), jnp.float32),
    )(parts, degs3, W, b.reshape(1, H))


def kernel(x, edge_index, W1, b1, W2, b2, size):
    N, D = x.shape
    E = edge_index.shape[1]
    H = W1.shape[1]

    nchunk = _round_up(E, NC * NS * CHUNK * 2 * KB) // (NC * NS * CHUNK)
    e_pad = NC * NS * nchunk * CHUNK
    n_pad = _round_up(N + 1, 1024)

    row = edge_index[0]
    col = edge_index[1]
    pad = e_pad - E
    dummy = N + jnp.arange(pad, dtype=jnp.int32) % (n_pad - N)
    rowp = jnp.concatenate([row, dummy])
    colp = jnp.concatenate([col, jnp.zeros((pad,), jnp.int32)])
    rowh = rowp.reshape(NC, NS, nchunk, CHUNK)
    colh = colp.reshape(NC, NS, nchunk, CHUNK)

    rpt = n_pad // NS
    zeros2d = jnp.zeros((rpt, D), jnp.float32)
    zeros1d = jnp.zeros((rpt,), jnp.float32)
    ones1 = jnp.ones((CHUNK,), jnp.float32)

    agg1, deg1 = _sc_aggregate(x, colh, rowh, zeros2d, zeros1d, ones1,
                               n_pad, nchunk)
    h = _tc_linear(agg1, deg1.reshape(NC, n_pad, 1), W1, b1, relu=True)
    agg2, deg2 = _sc_aggregate(h, colh, rowh, zeros2d, zeros1d, ones1,
                               n_pad, nchunk)
    logits = _tc_linear(agg2, deg2.reshape(NC, n_pad, 1), W2, b2, relu=False)
    return logits[:N]


# speedup vs baseline: 1.1813x; 1.1813x over previous
import functools

import jax
import jax.numpy as jnp
from jax import lax
from jax.experimental import pallas as pl
from jax.experimental.pallas import tpu as pltpu
from jax.experimental.pallas import tpu_sc as plsc

NC = 2
NS = 16
CHUNK = 128
KB = 8


def _round_up(a, b):
    return (a + b - 1) // b * b


def _sc_aggregate(data, colh, rowh, zeros2d, zeros1d, ones1, n_pad,
                  nch0, nch1, with_deg):
    D = data.shape[1]
    rpt = n_pad // NS
    mesh = plsc.VectorSubcoreMesh(core_axis_name="c", subcore_axis_name="s")

    out_type = [jax.ShapeDtypeStruct((NC, n_pad, D), jnp.float32)]
    deg_scratch = []
    if with_deg:
        out_type.append(jax.ShapeDtypeStruct((NC, n_pad), jnp.float32))
        deg_scratch = [pltpu.VMEM_SHARED((n_pad,), jnp.float32)]

    @functools.partial(
        pl.kernel,
        out_type=tuple(out_type),
        mesh=mesh,
        scratch_types=[
            pltpu.VMEM((2, KB, CHUNK), jnp.int32),
            pltpu.VMEM((2, KB, CHUNK), jnp.int32),
            pltpu.VMEM((2, CHUNK, D), jnp.float32),
            pltpu.VMEM((CHUNK,), jnp.float32),
            pltpu.VMEM_SHARED((n_pad, D), jnp.float32),
            *deg_scratch,
            pltpu.SemaphoreType.DMA,
            pltpu.SemaphoreType.DMA,
        ],
    )
    def k(data_hbm, col_hbm, row_hbm, z2_hbm, z1_hbm, ones_hbm, *rest):
        if with_deg:
            (agg_out, deg_out, col_w, row_w, gbuf, ones_v, sh_agg, sh_deg,
             gsem, isem) = rest
        else:
            agg_out, col_w, row_w, gbuf, ones_v, sh_agg, gsem, isem = rest
        c = lax.axis_index("c")
        s = lax.axis_index("s")
        rbase = s * rpt
        pltpu.sync_copy(z2_hbm, sh_agg.at[pl.ds(rbase, rpt)])
        if with_deg:
            pltpu.sync_copy(z1_hbm, sh_deg.at[pl.ds(rbase, rpt)])
            pltpu.sync_copy(ones_hbm, ones_v)
        plsc.subcore_barrier()

        def run(base, nch):
            nw = nch // KB
            pltpu.async_copy(col_hbm.at[pl.ds(base, KB)], col_w.at[0], isem)
            pltpu.async_copy(row_hbm.at[pl.ds(base, KB)], row_w.at[0], isem)

            @pl.loop(0, nw, step=2)
            def _(ww):
                for slot in range(2):
                    w = ww + slot
                    wb = base + w * KB
                    pltpu.make_async_copy(
                        col_hbm.at[pl.ds(wb, KB)], col_w.at[slot], isem).wait()
                    pltpu.make_async_copy(
                        row_hbm.at[pl.ds(wb, KB)], row_w.at[slot], isem).wait()

                    @pl.when(w + 1 < nw)
                    def _():
                        nb = base + (w + 1) * KB
                        pltpu.async_copy(col_hbm.at[pl.ds(nb, KB)],
                                         col_w.at[1 - slot], isem)
                        pltpu.async_copy(row_hbm.at[pl.ds(nb, KB)],
                                         row_w.at[1 - slot], isem)

                    @pl.loop(0, KB, step=2)
                    def _(jj):
                        da = pltpu.async_copy(
                            data_hbm.at[col_w.at[slot, jj]], gbuf.at[0], gsem)
                        db = pltpu.async_copy(
                            data_hbm.at[col_w.at[slot, jj + 1]], gbuf.at[1],
                            gsem)
                        da.wait()
                        pltpu.sync_copy(gbuf.at[0],
                                        sh_agg.at[row_w.at[slot, jj]],
                                        add=True)
                        if with_deg:
                            pltpu.sync_copy(ones_v,
                                            sh_deg.at[row_w.at[slot, jj]],
                                            add=True)
                        db.wait()
                        pltpu.sync_copy(gbuf.at[1],
                                        sh_agg.at[row_w.at[slot, jj + 1]],
                                        add=True)
                        if with_deg:
                            pltpu.sync_copy(ones_v,
                                            sh_deg.at[row_w.at[slot, jj + 1]],
                                            add=True)

        @pl.when(c == 0)
        def _():
            run(s * nch0, nch0)

        @pl.when(c == 1)
        def _():
            run(NS * nch0 + s * nch1, nch1)

        plsc.subcore_barrier()
        pltpu.sync_copy(sh_agg.at[pl.ds(rbase, rpt)],
                        agg_out.at[c, pl.ds(rbase, rpt)])
        if with_deg:
            pltpu.sync_copy(sh_deg.at[pl.ds(rbase, rpt)],
                            deg_out.at[c, pl.ds(rbase, rpt)])

    return k(data, colh, rowh, zeros2d, zeros1d, ones1)


def _tc_linear(parts, degs3, W, b, relu):
    n_pad, D = parts.shape[1], parts.shape[2]
    H = W.shape[1]
    BLK = 512

    def body(p_ref, d_ref, w_ref, b_ref, o_ref):
        agg = p_ref[0] + p_ref[1]
        deg = d_ref[0] + d_ref[1]
        dinv = jnp.where(deg > 0, lax.rsqrt(jnp.maximum(deg, 1.0)), 0.0)
        out = jnp.dot(agg * dinv, w_ref[...],
                      preferred_element_type=jnp.float32,
                      precision=lax.Precision.HIGHEST)
        out = out + b_ref[...]
        if relu:
            out = jnp.maximum(out, 0.0)
        o_ref[...] = out

    return pl.pallas_call(
        body,
        grid=(n_pad // BLK,),
        in_specs=[
            pl.BlockSpec((NC, BLK, D), lambda i: (0, i, 0)),
            pl.BlockSpec((NC, BLK, 1), lambda i: (0, i, 0)),
            pl.BlockSpec((D, H), lambda i: (0, 0)),
            pl.BlockSpec((1, H), lambda i: (0, 0)),
        ],
        out_specs=pl.BlockSpec((BLK, H), lambda i: (i, 0)),
        out_shape=jax.ShapeDtypeStruct((n_pad, H), jnp.float32),
    )(parts, degs3, W, b.reshape(1, H))


def kernel(x, edge_index, W1, b1, W2, b2, size):
    N, D = x.shape
    E = edge_index.shape[1]
    H = W1.shape[1]

    tot = _round_up(E, NS * CHUNK * 4 * KB) // (NS * CHUNK)
    nch1 = max(2 * KB, (tot // 5) // (2 * KB) * (2 * KB))
    nch0 = tot - nch1
    e_pad = NS * tot * CHUNK
    n_pad = _round_up(N + 1, 1024)

    row = edge_index[0]
    col = edge_index[1]
    pad = e_pad - E
    dummy = N + jnp.arange(pad, dtype=jnp.int32) % (n_pad - N)
    rowp = jnp.concatenate([row, dummy])
    colp = jnp.concatenate([col, jnp.zeros((pad,), jnp.int32)])
    rowh = rowp.reshape(NS * tot, CHUNK)
    colh = colp.reshape(NS * tot, CHUNK)

    rpt = n_pad // NS
    zeros2d = jnp.zeros((rpt, D), jnp.float32)
    zeros1d = jnp.zeros((rpt,), jnp.float32)
    ones1 = jnp.ones((CHUNK,), jnp.float32)

    agg1, deg = _sc_aggregate(x, colh, rowh, zeros2d, zeros1d, ones1,
                              n_pad, nch0, nch1, with_deg=True)
    degs3 = deg.reshape(NC, n_pad, 1)
    h = _tc_linear(agg1, degs3, W1, b1, relu=True)
    (agg2,) = _sc_aggregate(h, colh, rowh, zeros2d, zeros1d, ones1,
                            n_pad, nch0, nch1, with_deg=False)
    logits = _tc_linear(agg2, degs3, W2, b2, relu=False)
    return logits[:N]


# speedup vs baseline: 1.1821x; 1.0007x over previous
import functools

import jax
import jax.numpy as jnp
from jax import lax
from jax.experimental import pallas as pl
from jax.experimental.pallas import tpu as pltpu
from jax.experimental.pallas import tpu_sc as plsc

NC = 2
NS = 16
CHUNK = 128
KB = 8


def _round_up(a, b):
    return (a + b - 1) // b * b


def _sc_aggregate(data, colh, rowh, zeros2d, zeros1d, ones1, n_pad,
                  nch0, nch1, with_deg):
    D = data.shape[1]
    rpt = n_pad // NS
    mesh = plsc.VectorSubcoreMesh(core_axis_name="c", subcore_axis_name="s")

    out_type = [jax.ShapeDtypeStruct((NC, n_pad, D), jnp.float32)]
    deg_scratch = []
    if with_deg:
        out_type.append(jax.ShapeDtypeStruct((NC, n_pad), jnp.float32))
        deg_scratch = [pltpu.VMEM_SHARED((n_pad,), jnp.float32)]

    @functools.partial(
        pl.kernel,
        out_type=tuple(out_type),
        mesh=mesh,
        scratch_types=[
            pltpu.VMEM((2, KB, CHUNK), jnp.int32),
            pltpu.VMEM((2, KB, CHUNK), jnp.int32),
            pltpu.VMEM((2, CHUNK, D), jnp.float32),
            pltpu.VMEM((CHUNK,), jnp.float32),
            pltpu.VMEM_SHARED((n_pad, D), jnp.float32),
            *deg_scratch,
            pltpu.SemaphoreType.DMA,
            pltpu.SemaphoreType.DMA,
        ],
    )
    def k(data_hbm, col_hbm, row_hbm, z2_hbm, z1_hbm, ones_hbm, *rest):
        if with_deg:
            (agg_out, deg_out, col_w, row_w, gbuf, ones_v, sh_agg, sh_deg,
             gsem, isem) = rest
        else:
            agg_out, col_w, row_w, gbuf, ones_v, sh_agg, gsem, isem = rest
        c = lax.axis_index("c")
        s = lax.axis_index("s")
        rbase = s * rpt
        with jax.named_scope("zero"):
            pltpu.sync_copy(z2_hbm, sh_agg.at[pl.ds(rbase, rpt)])
            if with_deg:
                pltpu.sync_copy(z1_hbm, sh_deg.at[pl.ds(rbase, rpt)])
                pltpu.sync_copy(ones_hbm, ones_v)
            plsc.subcore_barrier()

        def run(base, nch):
            nw = nch // KB
            pltpu.async_copy(col_hbm.at[pl.ds(base, KB)], col_w.at[0], isem)
            pltpu.async_copy(row_hbm.at[pl.ds(base, KB)], row_w.at[0], isem)

            @pl.loop(0, nw, step=2)
            def _(ww):
                for slot in range(2):
                    w = ww + slot
                    wb = base + w * KB
                    pltpu.make_async_copy(
                        col_hbm.at[pl.ds(wb, KB)], col_w.at[slot], isem).wait()
                    pltpu.make_async_copy(
                        row_hbm.at[pl.ds(wb, KB)], row_w.at[slot], isem).wait()

                    @pl.when(w + 1 < nw)
                    def _():
                        nb = base + (w + 1) * KB
                        pltpu.async_copy(col_hbm.at[pl.ds(nb, KB)],
                                         col_w.at[1 - slot], isem)
                        pltpu.async_copy(row_hbm.at[pl.ds(nb, KB)],
                                         row_w.at[1 - slot], isem)

                    @pl.loop(0, KB, step=2)
                    def _(jj):
                        da = pltpu.async_copy(
                            data_hbm.at[col_w.at[slot, jj]], gbuf.at[0], gsem)
                        db = pltpu.async_copy(
                            data_hbm.at[col_w.at[slot, jj + 1]], gbuf.at[1],
                            gsem)
                        da.wait()
                        pltpu.sync_copy(gbuf.at[0],
                                        sh_agg.at[row_w.at[slot, jj]],
                                        add=True)
                        if with_deg:
                            pltpu.sync_copy(ones_v,
                                            sh_deg.at[row_w.at[slot, jj]],
                                            add=True)
                        db.wait()
                        pltpu.sync_copy(gbuf.at[1],
                                        sh_agg.at[row_w.at[slot, jj + 1]],
                                        add=True)
                        if with_deg:
                            pltpu.sync_copy(ones_v,
                                            sh_deg.at[row_w.at[slot, jj + 1]],
                                            add=True)

        with jax.named_scope("edges"):
            @pl.when(c == 0)
            def _():
                run(s * nch0, nch0)

            @pl.when(c == 1)
            def _():
                run(NS * nch0 + s * nch1, nch1)

            plsc.subcore_barrier()

        with jax.named_scope("flush"):
            pltpu.sync_copy(sh_agg.at[pl.ds(rbase, rpt)],
                            agg_out.at[c, pl.ds(rbase, rpt)])
            if with_deg:
                pltpu.sync_copy(sh_deg.at[pl.ds(rbase, rpt)],
                                deg_out.at[c, pl.ds(rbase, rpt)])

    return k(data, colh, rowh, zeros2d, zeros1d, ones1)


def _tc_linear(parts, degs3, W, b, relu):
    n_pad, D = parts.shape[1], parts.shape[2]
    H = W.shape[1]
    BLK = 512

    def body(p_ref, d_ref, w_ref, b_ref, o_ref):
        agg = p_ref[0] + p_ref[1]
        deg = d_ref[0] + d_ref[1]
        dinv = jnp.where(deg > 0, lax.rsqrt(jnp.maximum(deg, 1.0)), 0.0)
        out = jnp.dot(agg * dinv, w_ref[...],
                      preferred_element_type=jnp.float32,
                      precision=lax.Precision.HIGHEST)
        out = out + b_ref[...]
        if relu:
            out = jnp.maximum(out, 0.0)
        o_ref[...] = out

    return pl.pallas_call(
        body,
        grid=(n_pad // BLK,),
        in_specs=[
            pl.BlockSpec((NC, BLK, D), lambda i: (0, i, 0)),
            pl.BlockSpec((NC, BLK, 1), lambda i: (0, i, 0)),
            pl.BlockSpec((D, H), lambda i: (0, 0)),
            pl.BlockSpec((1, H), lambda i: (0, 0)),
        ],
        out_specs=pl.BlockSpec((BLK, H), lambda i: (i, 0)),
        out_shape=jax.ShapeDtypeStruct((n_pad, H), jnp.float32),
    )(parts, degs3, W, b.reshape(1, H))


def kernel(x, edge_index, W1, b1, W2, b2, size):
    N, D = x.shape
    E = edge_index.shape[1]
    H = W1.shape[1]

    tot = _round_up(E, NS * CHUNK * 4 * KB) // (NS * CHUNK)
    nch1 = max(2 * KB, (tot // 5) // (2 * KB) * (2 * KB))
    nch0 = tot - nch1
    e_pad = NS * tot * CHUNK
    n_pad = _round_up(N + 1, 1024)

    row = edge_index[0]
    col = edge_index[1]
    pad = e_pad - E
    dummy = N + jnp.arange(pad, dtype=jnp.int32) % (n_pad - N)
    rowp = jnp.concatenate([row, dummy])
    colp = jnp.concatenate([col, jnp.zeros((pad,), jnp.int32)])
    rowh = rowp.reshape(NS * tot, CHUNK)
    colh = colp.reshape(NS * tot, CHUNK)

    rpt = n_pad // NS
    zeros2d = jnp.zeros((rpt, D), jnp.float32)
    zeros1d = jnp.zeros((rpt,), jnp.float32)
    ones1 = jnp.ones((CHUNK,), jnp.float32)

    agg1, deg = _sc_aggregate(x, colh, rowh, zeros2d, zeros1d, ones1,
                              n_pad, nch0, nch1, with_deg=True)
    degs3 = deg.reshape(NC, n_pad, 1)
    h = _tc_linear(agg1, degs3, W1, b1, relu=True)
    (agg2,) = _sc_aggregate(h, colh, rowh, zeros2d, zeros1d, ones1,
                            n_pad, nch0, nch1, with_deg=False)
    logits = _tc_linear(agg2, degs3, W2, b2, relu=False)
    return logits[:N]
